# Initial kernel scaffold; baseline (speedup 1.0000x reference)
import jax
import jax.numpy as jnp
from jax.experimental import pallas as pl


def kernel(feature, _target, anchor):
    raise NotImplementedError("write your pallas kernel here")


# speedup vs baseline: 9.1934x; 9.1934x over previous
import functools

import jax
import jax.numpy as jnp
from jax.experimental import pallas as pl
from jax.experimental.pallas import tpu as pltpu

CLS = 100
F = 128
B = 16384
RB = 2048
NBLK = B // RB
CP = 128
LAMBDA = 0.1


def _body(t_ref, f_ref, a_ref, out_ref, sacc, s2acc, cntacc):
    b = pl.program_id(0)

    @pl.when(b == 0)
    def _init():
        sacc[...] = jnp.zeros_like(sacc)
        s2acc[...] = jnp.zeros_like(s2acc)
        cntacc[...] = jnp.zeros_like(cntacc)

    fblk = f_ref[...]
    idx = t_ref[0, 0, :].astype(jnp.int32)
    cls_iota = jax.lax.broadcasted_iota(jnp.int32, (RB, CP), 1)
    onehot = (idx[:, None] == cls_iota).astype(jnp.float32)

    sacc[...] += jax.lax.dot_general(
        onehot, fblk, (((0,), (0,)), ((), ())),
        preferred_element_type=jnp.float32,
        precision=jax.lax.Precision.HIGHEST)
    rowsq = jnp.sum(fblk * fblk, axis=1, keepdims=True)
    s2acc[...] += jax.lax.dot_general(
        onehot, rowsq, (((0,), (0,)), ((), ())),
        preferred_element_type=jnp.float32,
        precision=jax.lax.Precision.HIGHEST)
    cntacc[...] += jnp.sum(onehot, axis=0, keepdims=True)

    @pl.when(b == NBLK - 1)
    def _fin():
        a = a_ref[...]
        s = sacc[...]
        adots = jnp.sum(a * s, axis=1)
        asq = jnp.sum(a * a, axis=1)
        cnt = cntacc[0, :]
        s2 = s2acc[:, 0]
        good = cnt > 0.0
        contrib = jnp.where(
            good, (s2 - 2.0 * adots) / jnp.where(good, cnt, 1.0) + asq, 0.0)
        out_ref[...] = jnp.full((1, 1), LAMBDA * jnp.sum(contrib) / CLS,
                                dtype=jnp.float32)


@functools.partial(jax.jit, static_argnames=())
def kernel(feature, _target, anchor):
    t2 = _target.reshape(NBLK, 1, RB)
    a_pad = jnp.pad(anchor, ((0, CP - CLS), (0, 0)))
    out = pl.pallas_call(
        _body,
        grid=(NBLK,),
        in_specs=[
            pl.BlockSpec((1, 1, RB), lambda b: (b, 0, 0)),
            pl.BlockSpec((RB, F), lambda b: (b, 0)),
            pl.BlockSpec((CP, F), lambda b: (0, 0)),
        ],
        out_specs=pl.BlockSpec((1, 1), lambda b: (0, 0)),
        out_shape=jax.ShapeDtypeStruct((1, 1), jnp.float32),
        scratch_shapes=[
            pltpu.VMEM((CP, F), jnp.float32),
            pltpu.VMEM((CP, 1), jnp.float32),
            pltpu.VMEM((1, CP), jnp.float32),
        ],
    )(t2, feature, a_pad)
    return out[0, 0]
